# baseline (speedup 1.0000x reference)
import jax
import jax.numpy as jnp
from jax.experimental import pallas as pl
from jax.experimental.pallas import tpu as pltpu

_EPS = 1e-8
_LANES = 128
_BJ = 128


def _contrib_body(trow_ref, rrow_ref, tcol_ref, rcol_ref, ecol_ref, out_ref):
    trow = trow_ref[...]
    rrow = rrow_ref[...]
    tj = tcol_ref[...]
    rj = rcol_ref[...]
    ej = ecol_ref[...]

    ni = trow.shape[0]
    mx = jnp.max(rrow, keepdims=True)
    ex = jnp.exp(rrow - mx)
    tjb = jnp.broadcast_to(tj, (_BJ, _LANES))

    acc = jnp.zeros((_BJ, _LANES), jnp.float32)
    for c in range(ni):
        m = trow[c : c + 1, :] >= tjb
        acc = acc + jnp.where(m, ex[c : c + 1, :], 0.0)

    rss = jnp.sum(acc, axis=1, keepdims=True)
    log_loss = jnp.log(rss + _EPS) + mx
    out_ref[...] = (rj - log_loss) * ej


def _finish_body(c_ref, e_ref, o_ref):
    s_c = jnp.sum(jnp.sum(c_ref[...], axis=1, keepdims=True), axis=0, keepdims=True)
    s_e = jnp.sum(jnp.sum(e_ref[...], axis=1, keepdims=True), axis=0, keepdims=True)
    n = jnp.float32(c_ref.shape[0] * c_ref.shape[1])
    o_ref[...] = -s_c / (s_e * n)


def kernel(risk_pred, y_true):
    n = risk_pred.shape[0]
    ni = n // _LANES
    t = y_true[:, 0]
    e = y_true[:, 1]
    r = risk_pred.reshape(-1)

    trow = t.reshape(ni, _LANES)
    rrow = r.reshape(ni, _LANES)
    tcol = t.reshape(n, 1)
    rcol = r.reshape(n, 1)
    ecol = e.reshape(n, 1)

    contrib = pl.pallas_call(
        _contrib_body,
        grid=(n // _BJ,),
        in_specs=[
            pl.BlockSpec((ni, _LANES), lambda i: (0, 0)),
            pl.BlockSpec((ni, _LANES), lambda i: (0, 0)),
            pl.BlockSpec((_BJ, 1), lambda i: (i, 0)),
            pl.BlockSpec((_BJ, 1), lambda i: (i, 0)),
            pl.BlockSpec((_BJ, 1), lambda i: (i, 0)),
        ],
        out_specs=pl.BlockSpec((_BJ, 1), lambda i: (i, 0)),
        out_shape=jax.ShapeDtypeStruct((n, 1), jnp.float32),
        compiler_params=pltpu.CompilerParams(
            dimension_semantics=("parallel",),
        ),
    )(trow, rrow, tcol, rcol, ecol)

    nll = pl.pallas_call(
        _finish_body,
        out_shape=jax.ShapeDtypeStruct((1, 1), jnp.float32),
    )(contrib.reshape(ni, _LANES), e.reshape(ni, _LANES))
    return nll[0, 0]


# speedup vs baseline: 1.9977x; 1.9977x over previous
import jax
import jax.numpy as jnp
from jax.experimental import pallas as pl
from jax.experimental.pallas import tpu as pltpu

_EPS = 1e-8
_LANES = 128
_SUB = 8


def _nll_body(trow_ref, rrow_ref, erow_ref, o_ref, acc_ref):
    pid = pl.program_id(0)
    ncells = pl.num_programs(0)

    trow = trow_ref[...]
    rrow = rrow_ref[...]
    ni = rrow.shape[0]

    mx = jnp.max(rrow, keepdims=True)
    ex = jnp.exp(rrow - mx)

    row0 = pl.multiple_of(pid * _SUB, _SUB)
    t8 = trow_ref[pl.ds(row0, _SUB), :]
    tT = jnp.transpose(t8)

    cols = []
    for c in range(_SUB):
        tjb = jnp.broadcast_to(tT[:, c : c + 1], (_LANES, _LANES))
        acc = jnp.zeros((_LANES, _LANES), jnp.float32)
        for ci in range(ni):
            m = trow[ci : ci + 1, :] >= tjb
            acc = acc + jnp.where(m, ex[ci : ci + 1, :], 0.0)
        cols.append(jnp.sum(acc, axis=1, keepdims=True))

    rss = jnp.transpose(jnp.concatenate(cols, axis=1))
    log_loss = jnp.log(rss + _EPS) + mx
    r8 = rrow_ref[pl.ds(row0, _SUB), :]
    e8 = erow_ref[pl.ds(row0, _SUB), :]
    contrib = (r8 - log_loss) * e8

    @pl.when(pid == 0)
    def _():
        acc_ref[...] = jnp.zeros_like(acc_ref)

    acc_ref[...] += contrib

    @pl.when(pid == ncells - 1)
    def _():
        erow = erow_ref[...]
        s_c = jnp.sum(jnp.sum(acc_ref[...], axis=1, keepdims=True), axis=0, keepdims=True)
        s_e = jnp.sum(jnp.sum(erow, axis=1, keepdims=True), axis=0, keepdims=True)
        n = jnp.float32(ni * _LANES)
        o_ref[...] = -s_c / (s_e * n)


def kernel(risk_pred, y_true):
    n = risk_pred.shape[0]
    ni = n // _LANES
    trow = y_true[:, 0].reshape(ni, _LANES)
    erow = y_true[:, 1].reshape(ni, _LANES)
    rrow = risk_pred.reshape(ni, _LANES)

    nll = pl.pallas_call(
        _nll_body,
        grid=(ni // _SUB,),
        in_specs=[
            pl.BlockSpec((ni, _LANES), lambda i: (0, 0)),
            pl.BlockSpec((ni, _LANES), lambda i: (0, 0)),
            pl.BlockSpec((ni, _LANES), lambda i: (0, 0)),
        ],
        out_specs=pl.BlockSpec((1, 1), lambda i: (0, 0)),
        out_shape=jax.ShapeDtypeStruct((1, 1), jnp.float32),
        scratch_shapes=[pltpu.VMEM((_SUB, _LANES), jnp.float32)],
        compiler_params=pltpu.CompilerParams(
            dimension_semantics=("arbitrary",),
        ),
    )(trow, rrow, erow)
    return nll[0, 0]
